# Initial kernel scaffold; baseline (speedup 1.0000x reference)
import jax
import jax.numpy as jnp
from jax.experimental import pallas as pl


def kernel(query_times, event_times, mu, alpha, beta, norm_constants):
    raise NotImplementedError("write your pallas kernel here")


# speedup vs baseline: 2586.9499x; 2586.9499x over previous
import functools

import jax
import jax.numpy as jnp
from jax import lax
from jax.experimental import pallas as pl
from jax.experimental.pallas import tpu as pltpu
from jax.experimental.pallas import tpu_sc as plsc

B, P, L, M, L_EVAL = 8, 16, 256, 64, 2048
LANES = 16
NC_CORES = 2
NS = 16
NW = NC_CORES * NS
PAIRS = B * P
PAIRS_PER_W = PAIRS // NW
QCHUNK = 512
NCHUNK = L_EVAL // QCHUNK
JV_PER_CHUNK = QCHUNK // LANES


def _sc_body(q_hbm, et_hbm, mu_hbm, al_hbm, be_hbm, nc_hbm, invnc_hbm,
             out_hbm,
             mu_v, al_v, be_v, et_v, q_v, nc_v, invnc_v, idx_v, ndt_v, outb):
    wid = lax.axis_index("s") * NC_CORES + lax.axis_index("c")

    def pair_body(k, carry):
        pid = wid * PAIRS_PER_W + k
        b = pid // P
        p = pid % P

        pltpu.sync_copy(et_hbm.at[b, p], et_v)
        pltpu.sync_copy(q_hbm.at[b, p], q_v)
        pltpu.sync_copy(mu_hbm.at[b, :, p, :], mu_v)
        pltpu.sync_copy(al_hbm.at[b, :, p, :], al_v)
        pltpu.sync_copy(be_hbm.at[b, :, p, :], be_v)
        pltpu.sync_copy(nc_hbm.at[b], nc_v)
        pltpu.sync_copy(invnc_hbm.at[b], invnc_v)

        ncv = nc_v[...]
        invncv = invnc_v[...]

        def search_body(jv, carry):
            q = q_v[pl.ds(jv * LANES, LANES)]
            qn = q / ncv
            pos = jnp.zeros((LANES,), jnp.int32)
            for sz in (128, 64, 32, 16, 8, 4, 2, 1):
                probe = pos + (sz - 1)
                vals = plsc.load_gather(et_v, [probe])
                pos = jnp.where(vals < qn, pos + sz, pos)
            vals = plsc.load_gather(et_v, [pos])
            ss = jnp.where(vals < qn, pos + 1, pos)
            idx = jnp.maximum(ss - 1, 0)
            tl = plsc.load_gather(et_v, [idx])
            tl = jnp.where(ss == 0, jnp.zeros((LANES,), jnp.float32), tl)
            idx_v[pl.ds(jv * LANES, LANES)] = idx
            ndt_v[pl.ds(jv * LANES, LANES)] = tl - qn
            return carry

        lax.fori_loop(0, L_EVAL // LANES, search_body, 0)

        def chunk_body(c, carry):
            def jv_body(jv, carry):
                qoff = c * QCHUNK + jv * LANES
                idxq = idx_v[pl.ds(qoff, LANES)]
                ndt = ndt_v[pl.ds(qoff, LANES)]
                for m in range(M):
                    mv = jnp.full((LANES,), m, jnp.int32)
                    g_mu = plsc.load_gather(mu_v, [mv, idxq])
                    g_al = plsc.load_gather(al_v, [mv, idxq])
                    g_be = plsc.load_gather(be_v, [mv, idxq])
                    e = jnp.exp(g_be * ndt)
                    res = (g_mu + (g_al - g_mu) * e) * invncv
                    outb[m, pl.ds(jv * LANES, LANES)] = res
                return carry

            lax.fori_loop(0, JV_PER_CHUNK, jv_body, 0)
            pltpu.sync_copy(outb, out_hbm.at[b, :, p, pl.ds(c * QCHUNK, QCHUNK)])
            return carry

        lax.fori_loop(0, NCHUNK, chunk_body, 0)
        return carry

    lax.fori_loop(0, PAIRS_PER_W, pair_body, 0)


def kernel(query_times, event_times, mu, alpha, beta, norm_constants):
    nc_b = jnp.broadcast_to(norm_constants[:, None], (B, LANES))
    invnc_b = jnp.broadcast_to((1.0 / norm_constants)[:, None], (B, LANES))

    mesh = plsc.VectorSubcoreMesh(core_axis_name="c", subcore_axis_name="s")
    run = pl.kernel(
        _sc_body,
        out_type=jax.ShapeDtypeStruct((B, M, P, L_EVAL), jnp.float32),
        mesh=mesh,
        compiler_params=pltpu.CompilerParams(needs_layout_passes=False),
        scratch_types=[
            pltpu.VMEM((M, L), jnp.float32),
            pltpu.VMEM((M, L), jnp.float32),
            pltpu.VMEM((M, L), jnp.float32),
            pltpu.VMEM((L,), jnp.float32),
            pltpu.VMEM((L_EVAL,), jnp.float32),
            pltpu.VMEM((LANES,), jnp.float32),
            pltpu.VMEM((LANES,), jnp.float32),
            pltpu.VMEM((L_EVAL,), jnp.int32),
            pltpu.VMEM((L_EVAL,), jnp.float32),
            pltpu.VMEM((M, QCHUNK), jnp.float32),
        ],
    )
    return run(query_times, event_times, mu, alpha, beta, nc_b, invnc_b)


# speedup vs baseline: 7737.8599x; 2.9911x over previous
import jax
import jax.numpy as jnp
from jax import lax
from jax.experimental import pallas as pl
from jax.experimental.pallas import tpu as pltpu
from jax.experimental.pallas import tpu_sc as plsc

B, P, L, M, L_EVAL = 8, 16, 256, 64, 2048
LANES = 16
NCORES = 2
NSUB = 16
NW = NCORES * NSUB
PAIRS = B * P
PAIRS_PER_W = PAIRS // NW
QCHUNK = 512
NCHUNK = L_EVAL // QCHUNK
JV_PER_CHUNK = QCHUNK // LANES
MGROUP = 8
SEARCH_WAY = 4


def _sc_body(q_hbm, et_hbm, mu_hbm, al_hbm, be_hbm, nc_hbm, invnc_hbm,
             out_hbm,
             mu_v, al_v, be_v, et_v, q_v, nc_v, invnc_v, idx_v, ndt_v,
             outb0, outb1, sem_p, sem_o0, sem_o1):
    wid = lax.axis_index("s") * NCORES + lax.axis_index("c")

    def drain(buf, sem):
        pltpu.make_async_copy(
            out_hbm.at[0, :, 0, pl.ds(0, QCHUNK)], buf, sem).wait()

    def pair_body(k, carry):
        pid = wid * PAIRS_PER_W + k
        b = pid // P
        p = pid % P

        cp_mu = pltpu.async_copy(mu_hbm.at[b, :, p, :], mu_v, sem_p)
        cp_al = pltpu.async_copy(al_hbm.at[b, :, p, :], al_v, sem_p)
        cp_be = pltpu.async_copy(be_hbm.at[b, :, p, :], be_v, sem_p)
        pltpu.sync_copy(et_hbm.at[b, p], et_v)
        pltpu.sync_copy(q_hbm.at[b, p], q_v)
        pltpu.sync_copy(nc_hbm.at[b], nc_v)
        pltpu.sync_copy(invnc_hbm.at[b], invnc_v)

        ncv = nc_v[...]

        def search_body(jj, carry):
            qns = []
            poss = []
            for w in range(SEARCH_WAY):
                jv = jj * SEARCH_WAY + w
                q = q_v[pl.ds(jv * LANES, LANES)]
                qns.append(q / ncv)
                poss.append(jnp.zeros((LANES,), jnp.int32))
            for sz in (128, 64, 32, 16, 8, 4, 2, 1):
                vals = [plsc.load_gather(et_v, [poss[w] + (sz - 1)])
                        for w in range(SEARCH_WAY)]
                poss = [jnp.where(vals[w] < qns[w], poss[w] + sz, poss[w])
                        for w in range(SEARCH_WAY)]
            vals = [plsc.load_gather(et_v, [poss[w]])
                    for w in range(SEARCH_WAY)]
            sss = [jnp.where(vals[w] < qns[w], poss[w] + 1, poss[w])
                   for w in range(SEARCH_WAY)]
            idxs = [jnp.maximum(sss[w] - 1, 0) for w in range(SEARCH_WAY)]
            tls = [plsc.load_gather(et_v, [idxs[w]])
                   for w in range(SEARCH_WAY)]
            for w in range(SEARCH_WAY):
                jv = jj * SEARCH_WAY + w
                tl = jnp.where(sss[w] == 0,
                               jnp.zeros((LANES,), jnp.float32), tls[w])
                idx_v[pl.ds(jv * LANES, LANES)] = idxs[w]
                ndt_v[pl.ds(jv * LANES, LANES)] = tl - qns[w]
            return carry

        lax.fori_loop(0, (L_EVAL // LANES) // SEARCH_WAY, search_body, 0)

        cp_mu.wait()
        cp_al.wait()
        cp_be.wait()

        invncv = invnc_v[...]

        def make_jv_body(outb):
            def jv_body(jv, carry):
                qoff = jv * LANES
                idxq = idx_v[pl.ds(carry + qoff, LANES)]
                ndt = ndt_v[pl.ds(carry + qoff, LANES)]

                def gload(g):
                    ms = [jnp.full((LANES,), g * MGROUP + i, jnp.int32)
                          for i in range(MGROUP)]
                    g_mu = [plsc.load_gather(mu_v, [mv, idxq]) for mv in ms]
                    g_al = [plsc.load_gather(al_v, [mv, idxq]) for mv in ms]
                    g_be = [plsc.load_gather(be_v, [mv, idxq]) for mv in ms]
                    return g_mu, g_al, g_be

                loaded = gload(0)
                for g in range(M // MGROUP):
                    g_mu, g_al, g_be = loaded
                    if g + 1 < M // MGROUP:
                        loaded = gload(g + 1)
                    for i in range(MGROUP):
                        m = g * MGROUP + i
                        e = jnp.exp(g_be[i] * ndt)
                        res = (g_mu[i] + (g_al[i] - g_mu[i]) * e) * invncv
                        outb[m, pl.ds(qoff, LANES)] = res
                return carry

            return jv_body

        def cc_body(cc, carry):
            c0 = cc * 2
            pred = (k * NCHUNK + c0) > 0

            @pl.when(pred)
            def _():
                drain(outb0, sem_o0)

            lax.fori_loop(0, JV_PER_CHUNK, make_jv_body(outb0),
                          c0 * QCHUNK)
            pltpu.async_copy(
                outb0, out_hbm.at[b, :, p, pl.ds(c0 * QCHUNK, QCHUNK)],
                sem_o0)

            @pl.when(pred)
            def _():
                drain(outb1, sem_o1)

            lax.fori_loop(0, JV_PER_CHUNK, make_jv_body(outb1),
                          (c0 + 1) * QCHUNK)
            pltpu.async_copy(
                outb1,
                out_hbm.at[b, :, p, pl.ds((c0 + 1) * QCHUNK, QCHUNK)],
                sem_o1)
            return carry

        lax.fori_loop(0, NCHUNK // 2, cc_body, 0)
        return carry

    lax.fori_loop(0, PAIRS_PER_W, pair_body, 0)
    drain(outb0, sem_o0)
    drain(outb1, sem_o1)


def kernel(query_times, event_times, mu, alpha, beta, norm_constants):
    nc_b = jnp.broadcast_to(norm_constants[:, None], (B, LANES))
    invnc_b = jnp.broadcast_to((1.0 / norm_constants)[:, None], (B, LANES))

    mesh = plsc.VectorSubcoreMesh(core_axis_name="c", subcore_axis_name="s")
    run = pl.kernel(
        _sc_body,
        out_type=jax.ShapeDtypeStruct((B, M, P, L_EVAL), jnp.float32),
        mesh=mesh,
        compiler_params=pltpu.CompilerParams(needs_layout_passes=False),
        scratch_types=[
            pltpu.VMEM((M, L), jnp.float32),
            pltpu.VMEM((M, L), jnp.float32),
            pltpu.VMEM((M, L), jnp.float32),
            pltpu.VMEM((L,), jnp.float32),
            pltpu.VMEM((L_EVAL,), jnp.float32),
            pltpu.VMEM((LANES,), jnp.float32),
            pltpu.VMEM((LANES,), jnp.float32),
            pltpu.VMEM((L_EVAL,), jnp.int32),
            pltpu.VMEM((L_EVAL,), jnp.float32),
            pltpu.VMEM((M, QCHUNK), jnp.float32),
            pltpu.VMEM((M, QCHUNK), jnp.float32),
            pltpu.SemaphoreType.DMA,
            pltpu.SemaphoreType.DMA,
            pltpu.SemaphoreType.DMA,
        ],
    )
    return run(query_times, event_times, mu, alpha, beta, nc_b, invnc_b)
